# baseline (speedup 1.0000x reference)
import functools

import jax
import jax.numpy as jnp
from jax import lax
from jax.experimental import pallas as pl
from jax.experimental.pallas import tpu as pltpu
from jax.experimental.pallas import tpu_sc as plsc

EPS = 1e-07
BETA = 1.0

_NC = 2
_NS = 16
_BB = 96
_GRP = 9


def _colmax_body(x_ref, o_ref):
    i = pl.program_id(0)
    bm = jnp.max(x_ref[...], axis=0, keepdims=True)

    @pl.when(i == 0)
    def _():
        o_ref[...] = bm

    @pl.when(i > 0)
    def _():
        o_ref[...] = jnp.maximum(o_ref[...], bm)


def _tables_body(x_ref, gmax_ref, o_ref):
    c = pl.program_id(0)
    m = jnp.maximum(x_ref[...], 0.0) + EPS
    z = jnp.exp(BETA * m - gmax_ref[...])
    val = jnp.where(c < 2, z, z * m)
    o_ref[...] = val[None]


def _mlp1_body(x_ref, sc_ref, w1_ref, b1_ref, h_ref, s1_ref, s2_ref):
    i = pl.program_id(0)
    blk = sc_ref[...]
    den = jnp.concatenate([blk[0], blk[1]], axis=1)
    num = jnp.concatenate([blk[2], blk[3]], axis=1)
    feats = x_ref[...] + num / (den + 1e-16)
    h = jnp.dot(feats, w1_ref[...], preferred_element_type=jnp.float32,
                precision=lax.Precision.HIGHEST) + b1_ref[...]
    h_ref[...] = h
    s1 = jnp.sum(h, axis=0, keepdims=True)
    s2 = jnp.sum(h * h, axis=0, keepdims=True)

    @pl.when(i == 0)
    def _():
        s1_ref[...] = s1
        s2_ref[...] = s2

    @pl.when(i > 0)
    def _():
        s1_ref[...] = s1_ref[...] + s1
        s2_ref[...] = s2_ref[...] + s2


def _mlp2_body(n_rows, h_ref, s1_ref, s2_ref, gamma_ref, beta_ref, w2_ref,
               b2_ref, o_ref):
    mean = s1_ref[...] / n_rows
    var = s2_ref[...] / n_rows - mean * mean
    rstd = lax.rsqrt(var + 1e-5)
    scale = rstd * gamma_ref[...]
    shift = beta_ref[...] - mean * scale
    hn = jnp.maximum(h_ref[...] * scale + shift, 0.0)
    o_ref[...] = jnp.dot(hn, w2_ref[...], preferred_element_type=jnp.float32,
                         precision=lax.Precision.HIGHEST) + b2_ref[...]


def _make_sc_seg_sum(n, n_pad, nb):
    rps = n_pad // _NS
    grp = _GRP
    mesh = plsc.VectorSubcoreMesh(core_axis_name="c", subcore_axis_name="s")

    @functools.partial(
        pl.kernel, mesh=mesh,
        out_type=jax.ShapeDtypeStruct((4, n_pad, 128), jnp.float32),
        scratch_types=[
            pltpu.VMEM((grp, _BB), jnp.int32),
            pltpu.VMEM((grp, _BB), jnp.int32),
            pltpu.VMEM((grp, _BB), jnp.int32),
            pltpu.VMEM((grp, _BB), jnp.int32),
            pltpu.VMEM((_BB, 128), jnp.float32),
            pltpu.VMEM((_BB, 128), jnp.float32),
            pltpu.VMEM((_BB, 128), jnp.float32),
            pltpu.VMEM_SHARED((n_pad, 128), jnp.float32),
            pltpu.SemaphoreType.DMA,
            pltpu.SemaphoreType.DMA,
            pltpu.SemaphoreType.DMA,
            pltpu.SemaphoreType.DMA,
            pltpu.SemaphoreType.DMA,
            pltpu.SemaphoreType.DMA,
            pltpu.SemaphoreType.DMA,
            pltpu.SemaphoreType.DMA,
        ])
    def sc_seg_sum(t0, t1, t2, t3, src_i, dst_i, zeros, out,
                   sv0, dv0, sv1, dv1, rows0, rows1, rows2, acc,
                   sg0, sg1, sg2, sa0, sa1, sa2, si0, si1):
        s = lax.axis_index("s")
        c = lax.axis_index("c")
        ng = nb // grp

        def group_body(gg, sv, dv, si):
            pltpu.make_async_copy(src_i.at[s, 0], sv, si).wait()
            pltpu.make_async_copy(dst_i.at[s, 0], dv, si).wait()

            pltpu.async_copy(tbl_ref[0].at[sv.at[0]], rows0, sg0)
            pltpu.async_copy(tbl_ref[0].at[sv.at[1]], rows1, sg1)
            pltpu.async_copy(tbl_ref[0].at[sv.at[2]], rows2, sg2)

            @pl.loop(0, grp, step=3)
            def _(j):
                pltpu.make_async_copy(tbl_ref[0].at[sv.at[0]], rows0,
                                      sg0).wait()
                a0 = pltpu.async_copy(rows0, acc.at[dv.at[j]], sa0, add=True)
                pltpu.make_async_copy(tbl_ref[0].at[sv.at[1]], rows1,
                                      sg1).wait()
                a1 = pltpu.async_copy(rows1, acc.at[dv.at[j + 1]], sa1,
                                      add=True)
                pltpu.make_async_copy(tbl_ref[0].at[sv.at[2]], rows2,
                                      sg2).wait()
                a2 = pltpu.async_copy(rows2, acc.at[dv.at[j + 2]], sa2,
                                      add=True)
                a0.wait()

                @pl.when(j + 3 < grp)
                def _():
                    pltpu.async_copy(tbl_ref[0].at[sv.at[j + 3]], rows0, sg0)

                a1.wait()

                @pl.when(j + 4 < grp)
                def _():
                    pltpu.async_copy(tbl_ref[0].at[sv.at[j + 4]], rows1, sg1)

                a2.wait()

                @pl.when(j + 5 < grp)
                def _():
                    pltpu.async_copy(tbl_ref[0].at[sv.at[j + 5]], rows2, sg2)

            @pl.when(gg + 2 < ng)
            def _():
                pltpu.async_copy(src_i.at[s, gg + 2], sv, si)
                pltpu.async_copy(dst_i.at[s, gg + 2], dv, si)

        tbl_ref = [None]

        def do_chunk(tbl, out_slab):
            tbl_ref[0] = tbl
            pltpu.sync_copy(zeros, acc.at[pl.ds(s * rps, rps)])
            plsc.subcore_barrier()

            pltpu.async_copy(src_i.at[s, 0], sv0, si0)
            pltpu.async_copy(dst_i.at[s, 0], dv0, si0)
            pltpu.async_copy(src_i.at[s, 1], sv1, si1)
            pltpu.async_copy(dst_i.at[s, 1], dv1, si1)

            @pl.loop(0, ng, step=2)
            def _(g):
                group_body(g, sv0, dv0, si0)
                group_body(g + 1, sv1, dv1, si1)

            plsc.subcore_barrier()
            pltpu.sync_copy(acc.at[pl.ds(s * rps, rps)],
                            out_slab.at[pl.ds(s * rps, rps)])

        @pl.when(c == 0)
        def _():
            do_chunk(t0, out.at[0])
            do_chunk(t2, out.at[2])

        @pl.when(c == 1)
        def _():
            do_chunk(t1, out.at[1])
            do_chunk(t3, out.at[3])

    return sc_seg_sum


def kernel(node_feats, edge_index, W1, b1, gamma, beta_bn, W2, b2):
    n, d = node_feats.shape
    e = edge_index.shape[1]
    h_dim = W1.shape[1]
    dh = d // 2

    blk = 1000
    grid_n = n // blk

    quantum = _BB * _GRP * 2
    epw = quantum * -(-e // (quantum * _NS))
    e_pad = epw * _NS
    nb = epw // _BB
    n_pad = -(-(n + 1) // (8 * _NS)) * (8 * _NS)

    gmax_x = pl.pallas_call(
        _colmax_body,
        grid=(grid_n,),
        in_specs=[pl.BlockSpec((blk, d), lambda i: (i, 0))],
        out_specs=pl.BlockSpec((1, d), lambda i: (0, 0)),
        out_shape=jax.ShapeDtypeStruct((1, d), jnp.float32),
        compiler_params=pltpu.CompilerParams(
            dimension_semantics=("arbitrary",)),
    )(node_feats)
    gmax = BETA * (jnp.maximum(gmax_x, 0.0) + EPS)

    tables = pl.pallas_call(
        _tables_body,
        grid=(4, grid_n),
        in_specs=[
            pl.BlockSpec((blk, dh), lambda c, i: (i, c % 2)),
            pl.BlockSpec((1, dh), lambda c, i: (0, c % 2)),
        ],
        out_specs=pl.BlockSpec((1, blk, dh), lambda c, i: (c, i, 0)),
        out_shape=jax.ShapeDtypeStruct((4, n, dh), jnp.float32),
        compiler_params=pltpu.CompilerParams(
            dimension_semantics=("parallel", "parallel")),
    )(node_feats, gmax)
    t0, t1, t2, t3 = (tables[0], tables[1], tables[2], tables[3])

    src = edge_index[0].astype(jnp.int32)
    dst = edge_index[1].astype(jnp.int32)
    pad = e_pad - e
    src_p = jnp.concatenate([src, jnp.zeros((pad,), jnp.int32)])
    dst_p = jnp.concatenate([dst, jnp.full((pad,), n, jnp.int32)])
    src_i = src_p.reshape(_NS, nb // _GRP, _GRP, _BB)
    dst_i = dst_p.reshape(_NS, nb // _GRP, _GRP, _BB)
    zeros = jnp.zeros((n_pad // _NS, dh), jnp.float32)

    sc_out = _make_sc_seg_sum(n, n_pad, nb)(t0, t1, t2, t3, src_i, dst_i,
                                            zeros)

    h, s1, s2 = pl.pallas_call(
        _mlp1_body,
        grid=(grid_n,),
        in_specs=[
            pl.BlockSpec((blk, d), lambda i: (i, 0)),
            pl.BlockSpec((4, blk, dh), lambda i: (0, i, 0)),
            pl.BlockSpec((d, h_dim), lambda i: (0, 0)),
            pl.BlockSpec((1, h_dim), lambda i: (0, 0)),
        ],
        out_specs=[
            pl.BlockSpec((blk, h_dim), lambda i: (i, 0)),
            pl.BlockSpec((1, h_dim), lambda i: (0, 0)),
            pl.BlockSpec((1, h_dim), lambda i: (0, 0)),
        ],
        out_shape=[
            jax.ShapeDtypeStruct((n, h_dim), jnp.float32),
            jax.ShapeDtypeStruct((1, h_dim), jnp.float32),
            jax.ShapeDtypeStruct((1, h_dim), jnp.float32),
        ],
        compiler_params=pltpu.CompilerParams(
            dimension_semantics=("arbitrary",)),
    )(node_feats, sc_out, W1, b1.reshape(1, h_dim))

    out = pl.pallas_call(
        functools.partial(_mlp2_body, float(n)),
        grid=(grid_n,),
        in_specs=[
            pl.BlockSpec((blk, h_dim), lambda i: (i, 0)),
            pl.BlockSpec((1, h_dim), lambda i: (0, 0)),
            pl.BlockSpec((1, h_dim), lambda i: (0, 0)),
            pl.BlockSpec((1, h_dim), lambda i: (0, 0)),
            pl.BlockSpec((1, h_dim), lambda i: (0, 0)),
            pl.BlockSpec((h_dim, d), lambda i: (0, 0)),
            pl.BlockSpec((1, d), lambda i: (0, 0)),
        ],
        out_specs=pl.BlockSpec((blk, d), lambda i: (i, 0)),
        out_shape=jax.ShapeDtypeStruct((n, d), jnp.float32),
        compiler_params=pltpu.CompilerParams(
            dimension_semantics=("parallel",)),
    )(h, s1, s2, gamma.reshape(1, h_dim), beta_bn.reshape(1, h_dim), W2,
      b2.reshape(1, d))

    return out


# speedup vs baseline: 1.7554x; 1.7554x over previous
import functools

import jax
import jax.numpy as jnp
from jax import lax
from jax.experimental import pallas as pl
from jax.experimental.pallas import tpu as pltpu
from jax.experimental.pallas import tpu_sc as plsc

EPS = 1e-07
BETA = 1.0

_NC = 2
_NS = 16
_BB = 112
_GRP = 9


def _colmax_body(x_ref, o_ref):
    i = pl.program_id(0)
    bm = jnp.max(x_ref[...], axis=0, keepdims=True)

    @pl.when(i == 0)
    def _():
        o_ref[...] = bm

    @pl.when(i > 0)
    def _():
        o_ref[...] = jnp.maximum(o_ref[...], bm)


def _tables_body(x_ref, gmax_ref, o_ref):
    c = pl.program_id(0)
    m = jnp.maximum(x_ref[...], 0.0) + EPS
    z = jnp.exp(BETA * m - gmax_ref[...])
    val = jnp.where(c < 2, z, z * m)
    o_ref[...] = val[None]


def _mlp1_body(x_ref, sc_ref, w1_ref, b1_ref, h_ref, s1_ref, s2_ref):
    i = pl.program_id(0)
    blk = sc_ref[...]
    den = jnp.concatenate([blk[0], blk[1]], axis=1)
    num = jnp.concatenate([blk[2], blk[3]], axis=1)
    feats = x_ref[...] + num / (den + 1e-16)
    h = jnp.dot(feats, w1_ref[...], preferred_element_type=jnp.float32,
                precision=lax.Precision.HIGHEST) + b1_ref[...]
    h_ref[...] = h
    s1 = jnp.sum(h, axis=0, keepdims=True)
    s2 = jnp.sum(h * h, axis=0, keepdims=True)

    @pl.when(i == 0)
    def _():
        s1_ref[...] = s1
        s2_ref[...] = s2

    @pl.when(i > 0)
    def _():
        s1_ref[...] = s1_ref[...] + s1
        s2_ref[...] = s2_ref[...] + s2


def _mlp2_body(n_rows, h_ref, s1_ref, s2_ref, gamma_ref, beta_ref, w2_ref,
               b2_ref, o_ref):
    mean = s1_ref[...] / n_rows
    var = s2_ref[...] / n_rows - mean * mean
    rstd = lax.rsqrt(var + 1e-5)
    scale = rstd * gamma_ref[...]
    shift = beta_ref[...] - mean * scale
    hn = jnp.maximum(h_ref[...] * scale + shift, 0.0)
    o_ref[...] = jnp.dot(hn, w2_ref[...], preferred_element_type=jnp.float32,
                         precision=lax.Precision.HIGHEST) + b2_ref[...]


def _make_sc_seg_sum(n, n_pad, nb):
    rps = n_pad // _NS
    grp = _GRP
    mesh = plsc.VectorSubcoreMesh(core_axis_name="c", subcore_axis_name="s")

    @functools.partial(
        pl.kernel, mesh=mesh,
        out_type=jax.ShapeDtypeStruct((4, n_pad, 128), jnp.float32),
        scratch_types=[
            pltpu.VMEM((grp, _BB), jnp.int32),
            pltpu.VMEM((grp, _BB), jnp.int32),
            pltpu.VMEM((_BB, 128), jnp.float32),
            pltpu.VMEM((_BB, 128), jnp.float32),
            pltpu.VMEM((_BB, 128), jnp.float32),
            pltpu.VMEM_SHARED((n_pad, 128), jnp.float32),
            pltpu.SemaphoreType.DMA,
            pltpu.SemaphoreType.DMA,
            pltpu.SemaphoreType.DMA,
            pltpu.SemaphoreType.DMA,
            pltpu.SemaphoreType.DMA,
            pltpu.SemaphoreType.DMA,
        ])
    def sc_seg_sum(t0, t1, t2, t3, src_i, dst_i, zeros, out,
                   src_v, dst_v, rows0, rows1, rows2, acc,
                   sg0, sg1, sg2, sa0, sa1, sa2):
        s = lax.axis_index("s")
        c = lax.axis_index("c")

        def do_chunk(tbl, out_slab):
            pltpu.sync_copy(zeros, acc.at[pl.ds(s * rps, rps)])
            plsc.subcore_barrier()

            @pl.loop(0, nb // grp)
            def _(g):
                pltpu.sync_copy(src_i.at[s, g], src_v)
                pltpu.sync_copy(dst_i.at[s, g], dst_v)

                pltpu.async_copy(tbl.at[src_v.at[0]], rows0, sg0)
                pltpu.async_copy(tbl.at[src_v.at[1]], rows1, sg1)
                pltpu.async_copy(tbl.at[src_v.at[2]], rows2, sg2)

                @pl.loop(0, grp, step=3)
                def _(j):
                    pltpu.make_async_copy(tbl.at[src_v.at[0]], rows0,
                                          sg0).wait()
                    a0 = pltpu.async_copy(rows0, acc.at[dst_v.at[j]], sa0,
                                          add=True)
                    pltpu.make_async_copy(tbl.at[src_v.at[1]], rows1,
                                          sg1).wait()
                    a1 = pltpu.async_copy(rows1, acc.at[dst_v.at[j + 1]],
                                          sa1, add=True)
                    pltpu.make_async_copy(tbl.at[src_v.at[2]], rows2,
                                          sg2).wait()
                    a2 = pltpu.async_copy(rows2, acc.at[dst_v.at[j + 2]],
                                          sa2, add=True)
                    a0.wait()

                    @pl.when(j + 3 < grp)
                    def _():
                        pltpu.async_copy(tbl.at[src_v.at[j + 3]], rows0, sg0)

                    a1.wait()

                    @pl.when(j + 4 < grp)
                    def _():
                        pltpu.async_copy(tbl.at[src_v.at[j + 4]], rows1, sg1)

                    a2.wait()

                    @pl.when(j + 5 < grp)
                    def _():
                        pltpu.async_copy(tbl.at[src_v.at[j + 5]], rows2, sg2)

            plsc.subcore_barrier()
            pltpu.sync_copy(acc.at[pl.ds(s * rps, rps)],
                            out_slab.at[pl.ds(s * rps, rps)])

        @pl.when(c == 0)
        def _():
            do_chunk(t0, out.at[0])
            do_chunk(t2, out.at[2])

        @pl.when(c == 1)
        def _():
            do_chunk(t1, out.at[1])
            do_chunk(t3, out.at[3])

    return sc_seg_sum


def kernel(node_feats, edge_index, W1, b1, gamma, beta_bn, W2, b2):
    n, d = node_feats.shape
    e = edge_index.shape[1]
    h_dim = W1.shape[1]
    dh = d // 2

    blk = 1000
    grid_n = n // blk

    epw = _BB * _GRP * -(-e // (_BB * _GRP * _NS))
    e_pad = epw * _NS
    nb = epw // _BB
    n_pad = -(-(n + 1) // (8 * _NS)) * (8 * _NS)

    gmax_x = pl.pallas_call(
        _colmax_body,
        grid=(grid_n,),
        in_specs=[pl.BlockSpec((blk, d), lambda i: (i, 0))],
        out_specs=pl.BlockSpec((1, d), lambda i: (0, 0)),
        out_shape=jax.ShapeDtypeStruct((1, d), jnp.float32),
        compiler_params=pltpu.CompilerParams(
            dimension_semantics=("arbitrary",)),
    )(node_feats)
    gmax = BETA * (jnp.maximum(gmax_x, 0.0) + EPS)

    tables = pl.pallas_call(
        _tables_body,
        grid=(4, grid_n),
        in_specs=[
            pl.BlockSpec((blk, dh), lambda c, i: (i, c % 2)),
            pl.BlockSpec((1, dh), lambda c, i: (0, c % 2)),
        ],
        out_specs=pl.BlockSpec((1, blk, dh), lambda c, i: (c, i, 0)),
        out_shape=jax.ShapeDtypeStruct((4, n, dh), jnp.float32),
        compiler_params=pltpu.CompilerParams(
            dimension_semantics=("parallel", "parallel")),
    )(node_feats, gmax)
    t0, t1, t2, t3 = (tables[0], tables[1], tables[2], tables[3])

    src = edge_index[0].astype(jnp.int32)
    dst = edge_index[1].astype(jnp.int32)
    pad = e_pad - e
    src_p = jnp.concatenate([src, jnp.zeros((pad,), jnp.int32)])
    dst_p = jnp.concatenate([dst, jnp.full((pad,), n, jnp.int32)])
    src_i = src_p.reshape(_NS, nb // _GRP, _GRP, _BB)
    dst_i = dst_p.reshape(_NS, nb // _GRP, _GRP, _BB)
    zeros = jnp.zeros((n_pad // _NS, dh), jnp.float32)

    sc_out = _make_sc_seg_sum(n, n_pad, nb)(t0, t1, t2, t3, src_i, dst_i,
                                            zeros)

    h, s1, s2 = pl.pallas_call(
        _mlp1_body,
        grid=(grid_n,),
        in_specs=[
            pl.BlockSpec((blk, d), lambda i: (i, 0)),
            pl.BlockSpec((4, blk, dh), lambda i: (0, i, 0)),
            pl.BlockSpec((d, h_dim), lambda i: (0, 0)),
            pl.BlockSpec((1, h_dim), lambda i: (0, 0)),
        ],
        out_specs=[
            pl.BlockSpec((blk, h_dim), lambda i: (i, 0)),
            pl.BlockSpec((1, h_dim), lambda i: (0, 0)),
            pl.BlockSpec((1, h_dim), lambda i: (0, 0)),
        ],
        out_shape=[
            jax.ShapeDtypeStruct((n, h_dim), jnp.float32),
            jax.ShapeDtypeStruct((1, h_dim), jnp.float32),
            jax.ShapeDtypeStruct((1, h_dim), jnp.float32),
        ],
        compiler_params=pltpu.CompilerParams(
            dimension_semantics=("arbitrary",)),
    )(node_feats, sc_out, W1, b1.reshape(1, h_dim))

    out = pl.pallas_call(
        functools.partial(_mlp2_body, float(n)),
        grid=(grid_n,),
        in_specs=[
            pl.BlockSpec((blk, h_dim), lambda i: (i, 0)),
            pl.BlockSpec((1, h_dim), lambda i: (0, 0)),
            pl.BlockSpec((1, h_dim), lambda i: (0, 0)),
            pl.BlockSpec((1, h_dim), lambda i: (0, 0)),
            pl.BlockSpec((1, h_dim), lambda i: (0, 0)),
            pl.BlockSpec((h_dim, d), lambda i: (0, 0)),
            pl.BlockSpec((1, d), lambda i: (0, 0)),
        ],
        out_specs=pl.BlockSpec((blk, d), lambda i: (i, 0)),
        out_shape=jax.ShapeDtypeStruct((n, d), jnp.float32),
        compiler_params=pltpu.CompilerParams(
            dimension_semantics=("parallel",)),
    )(h, s1, s2, gamma.reshape(1, h_dim), beta_bn.reshape(1, h_dim), W2,
      b2.reshape(1, d))

    return out
